# Initial kernel scaffold; baseline (speedup 1.0000x reference)
import jax
import jax.numpy as jnp
from jax.experimental import pallas as pl


def kernel(VelPoints, VMM):
    raise NotImplementedError("write your pallas kernel here")


# speedup vs baseline: 25.2830x; 25.2830x over previous
import functools

import jax
import jax.numpy as jnp
import numpy as np
from jax import lax
from jax.experimental import pallas as pl
from jax.experimental.pallas import tpu as pltpu
from jax.experimental.pallas import tpu_sc as plsc

_BS, _K, _N = 16, 8, 128
_H, _W = 256, 256
_OH, _OW = 512, 256
_C = _BS * _K
_CHUNK_ROWS = 256
_CHUNK = _CHUNK_ROWS * _OW
_GROUP = 16


def _interp_body(t_ref, v_ref, vmin_ref, vmax_ref, out_ref):
    t = t_ref[:] / np.float32(1.0 / (_H - 1))
    ts = jnp.where(t > 0, t, jnp.float32(1e9))
    vmin = vmin_ref[:]
    vmax = vmax_ref[:]
    stepv = (vmax - vmin) / np.float32(_W - 1)
    v = (v_ref[:] - vmin) / stepv

    g = t.shape[0]
    tsb = ts[:, None, :]
    vb = v[:, None, :]
    tq3 = lax.broadcasted_iota(jnp.int32, (g, _H, _N), 1).astype(jnp.float32)
    pidx = lax.broadcasted_iota(jnp.int32, (g, _H, _N), 2)

    neg = jnp.float32(-3e38)
    pos = jnp.float32(3e38)
    big = jnp.int32(1 << 30)

    le = tsb <= tq3
    tlo = jnp.max(jnp.where(le, tsb, neg), axis=2)
    plo = jnp.max(jnp.where(le & (tsb == tlo[:, :, None]), pidx, -1), axis=2)
    vlo = jnp.sum(jnp.where(pidx == plo[:, :, None], vb, 0.0), axis=2)

    gt = tsb > tq3
    thi = jnp.min(jnp.where(gt, tsb, pos), axis=2)
    phi = jnp.min(jnp.where(gt & (tsb == thi[:, :, None]), pidx, big), axis=2)
    vhi = jnp.sum(jnp.where(pidx == phi[:, :, None], vb, 0.0), axis=2)

    tqv = lax.broadcasted_iota(jnp.int32, (g, _H), 1).astype(jnp.float32)
    f = vlo + ((tqv - tlo) / (thi - tlo)) * (vhi - vlo)
    f = jnp.where(plo < 0, vhi, f)
    f = jnp.where(phi >= big, vlo, f)
    out_ref[:] = jnp.clip(jnp.round(f), 0, _W - 1).astype(jnp.int32)


_interp_call = pl.pallas_call(
    _interp_body,
    grid=(_C // _GROUP,),
    in_specs=[
        pl.BlockSpec((_GROUP, _N), lambda i: (i, 0)),
        pl.BlockSpec((_GROUP, _N), lambda i: (i, 0)),
        pl.BlockSpec((_GROUP, 1), lambda i: (i, 0)),
        pl.BlockSpec((_GROUP, 1), lambda i: (i, 0)),
    ],
    out_specs=pl.BlockSpec((_GROUP, _H), lambda i: (i, 0)),
    out_shape=jax.ShapeDtypeStruct((_C, _H), jnp.int32),
)


def _paint_call(vi, const01):
    mesh = plsc.VectorSubcoreMesh(core_axis_name="c", subcore_axis_name="s")
    info = plsc.get_sparse_core_info()
    nc = info.num_cores
    nw = nc * info.num_subcores
    curves_per_w = _C // nw

    @functools.partial(
        pl.kernel,
        out_type=jax.ShapeDtypeStruct((_C, _OH * _OW), jnp.float32),
        mesh=mesh,
        compiler_params=pltpu.CompilerParams(needs_layout_passes=False),
        scratch_types=[
            pltpu.VMEM((_CHUNK,), jnp.float32),
            pltpu.VMEM((_H,), jnp.int32),
        ],
    )
    def body(vi_hbm, const_hbm, out_hbm, buf, viv):
        wid = lax.axis_index("s") * nc + lax.axis_index("c")
        pltpu.sync_copy(const_hbm, buf)

        def scatter_pass(r0, wa_odd, wa_even, wb_odd, wb_even):
            def body_j(j, carry):
                rloc = j * 16 + lax.iota(jnp.int32, 16)
                r = r0 + rloc
                m = lax.shift_right_logical(r, 1)
                is_odd = lax.bitwise_and(r, 1) == 1
                ya = jnp.where(is_odd, m, jnp.maximum(m - 1, 0))
                yb = jnp.where(is_odd, jnp.minimum(m + 1, _H - 1), m)
                wa = jnp.where(is_odd, jnp.float32(wa_odd), jnp.float32(wa_even))
                wb = jnp.where(is_odd, jnp.float32(wb_odd), jnp.float32(wb_even))
                ca = plsc.load_gather(viv, [ya])
                cb = plsc.load_gather(viv, [yb])
                plsc.addupdate_scatter(buf, [rloc * _OW + ca], wa)
                plsc.addupdate_scatter(buf, [rloc * _OW + cb], wb)
                return carry
            lax.fori_loop(0, _CHUNK_ROWS // 16, body_j, 0)

        def per_curve(k, carry):
            c = wid * curves_per_w + k
            pltpu.sync_copy(vi_hbm.at[c], viv)

            def per_chunk(h, carry2):
                r0 = h * _CHUNK_ROWS
                scatter_pass(r0, 0.675, 0.225, 0.225, 0.675)
                pltpu.sync_copy(buf, out_hbm.at[c, pl.ds(r0 * _OW, _CHUNK)])
                scatter_pass(r0, -0.675, -0.225, -0.225, -0.675)
                return carry2
            lax.fori_loop(0, _OH // _CHUNK_ROWS, per_chunk, 0)
            return carry
        lax.fori_loop(0, curves_per_w, per_curve, 0)

    return body(vi, const01)


def kernel(VelPoints, VMM):
    t = VelPoints[..., 0].reshape(_C, _N)
    v = VelPoints[..., 1].reshape(_C, _N)
    vmin = jnp.repeat(VMM[:, 0], _K).reshape(_C, 1)
    vmax = jnp.repeat(VMM[:, 1], _K).reshape(_C, 1)
    vi = _interp_call(t, v, vmin, vmax)
    const01 = jnp.full((_CHUNK,), 0.01, jnp.float32)
    out = _paint_call(vi, const01)
    return out.reshape(_BS, _K, _OH, _OW)


# speedup vs baseline: 37.2368x; 1.4728x over previous
import functools

import jax
import jax.numpy as jnp
import numpy as np
from jax import lax
from jax.experimental import pallas as pl
from jax.experimental.pallas import tpu as pltpu
from jax.experimental.pallas import tpu_sc as plsc

_BS, _K, _N = 16, 8, 128
_H, _W = 256, 256
_OH, _OW = 512, 256
_C = _BS * _K
_CHUNK_ROWS = 256
_CHUNK = _CHUNK_ROWS * _OW
_GROUP = 16


def _interp_body(t_ref, v_ref, vmin_ref, vmax_ref, out_ref):
    t = t_ref[:] / np.float32(1.0 / (_H - 1))
    ts = jnp.where(t > 0, t, jnp.float32(1e9))
    vmin = vmin_ref[:]
    vmax = vmax_ref[:]
    stepv = (vmax - vmin) / np.float32(_W - 1)
    v = (v_ref[:] - vmin) / stepv

    g = t.shape[0]
    tsb = ts[:, None, :]
    vb = v[:, None, :]
    tq3 = lax.broadcasted_iota(jnp.int32, (g, _H, _N), 1).astype(jnp.float32)
    pidx = lax.broadcasted_iota(jnp.int32, (g, _H, _N), 2)

    neg = jnp.float32(-3e38)
    pos = jnp.float32(3e38)
    big = jnp.int32(1 << 30)

    le = tsb <= tq3
    tlo = jnp.max(jnp.where(le, tsb, neg), axis=2)
    plo = jnp.max(jnp.where(le & (tsb == tlo[:, :, None]), pidx, -1), axis=2)
    vlo = jnp.sum(jnp.where(pidx == plo[:, :, None], vb, 0.0), axis=2)

    gt = tsb > tq3
    thi = jnp.min(jnp.where(gt, tsb, pos), axis=2)
    phi = jnp.min(jnp.where(gt & (tsb == thi[:, :, None]), pidx, big), axis=2)
    vhi = jnp.sum(jnp.where(pidx == phi[:, :, None], vb, 0.0), axis=2)

    tqv = lax.broadcasted_iota(jnp.int32, (g, _H), 1).astype(jnp.float32)
    f = vlo + ((tqv - tlo) / (thi - tlo)) * (vhi - vlo)
    f = jnp.where(plo < 0, vhi, f)
    f = jnp.where(phi >= big, vlo, f)
    out_ref[:] = jnp.clip(jnp.round(f), 0, _W - 1).astype(jnp.int32)


_interp_call = pl.pallas_call(
    _interp_body,
    grid=(_C // _GROUP,),
    in_specs=[
        pl.BlockSpec((_GROUP, _N), lambda i: (i, 0)),
        pl.BlockSpec((_GROUP, _N), lambda i: (i, 0)),
        pl.BlockSpec((_GROUP, 1), lambda i: (i, 0)),
        pl.BlockSpec((_GROUP, 1), lambda i: (i, 0)),
    ],
    out_specs=pl.BlockSpec((_GROUP, _H), lambda i: (i, 0)),
    out_shape=jax.ShapeDtypeStruct((_C, _H), jnp.int32),
)


def _paint_call(vi, const01):
    mesh = plsc.VectorSubcoreMesh(core_axis_name="c", subcore_axis_name="s")
    info = plsc.get_sparse_core_info()
    nc = info.num_cores
    nw = nc * info.num_subcores
    curves_per_w = _C // nw

    @functools.partial(
        pl.kernel,
        out_type=jax.ShapeDtypeStruct((_BS, _K, _OH, _OW), jnp.float32),
        mesh=mesh,
        compiler_params=pltpu.CompilerParams(needs_layout_passes=False),
        scratch_types=[
            pltpu.VMEM((_CHUNK_ROWS, _OW), jnp.float32),
            pltpu.VMEM((_H,), jnp.int32),
        ],
    )
    def body(vi_hbm, const_hbm, out_hbm, buf, viv):
        wid = lax.axis_index("s") * nc + lax.axis_index("c")
        pltpu.sync_copy(const_hbm, buf)

        def scatter_pass(r0, wa_odd, wa_even, wb_odd, wb_even):
            def body_j(j, carry):
                rloc = j * 16 + lax.iota(jnp.int32, 16)
                r = r0 + rloc
                m = lax.shift_right_logical(r, 1)
                is_odd = lax.bitwise_and(r, 1) == 1
                ya = jnp.where(is_odd, m, jnp.maximum(m - 1, 0))
                yb = jnp.where(is_odd, jnp.minimum(m + 1, _H - 1), m)
                wa = jnp.where(is_odd, jnp.float32(wa_odd), jnp.float32(wa_even))
                wb = jnp.where(is_odd, jnp.float32(wb_odd), jnp.float32(wb_even))
                ca = plsc.load_gather(viv, [ya])
                cb = plsc.load_gather(viv, [yb])
                plsc.addupdate_scatter(buf, [rloc, ca], wa)
                plsc.addupdate_scatter(buf, [rloc, cb], wb)
                return carry
            lax.fori_loop(0, _CHUNK_ROWS // 16, body_j, 0)

        def per_curve(k, carry):
            c = wid * curves_per_w + k
            b = lax.div(c, _K)
            kk = lax.rem(c, _K)
            pltpu.sync_copy(vi_hbm.at[c], viv)

            def per_chunk(h, carry2):
                r0 = h * _CHUNK_ROWS
                scatter_pass(r0, 0.675, 0.225, 0.225, 0.675)
                pltpu.sync_copy(buf, out_hbm.at[b, kk, pl.ds(r0, _CHUNK_ROWS)])
                scatter_pass(r0, -0.675, -0.225, -0.225, -0.675)
                return carry2
            lax.fori_loop(0, _OH // _CHUNK_ROWS, per_chunk, 0)
            return carry
        lax.fori_loop(0, curves_per_w, per_curve, 0)

    return body(vi, const01)


def kernel(VelPoints, VMM):
    t = VelPoints[..., 0].reshape(_C, _N)
    v = VelPoints[..., 1].reshape(_C, _N)
    vmin = jnp.repeat(VMM[:, 0], _K).reshape(_C, 1)
    vmax = jnp.repeat(VMM[:, 1], _K).reshape(_C, 1)
    vi = _interp_call(t, v, vmin, vmax)
    const01 = jnp.full((_CHUNK_ROWS, _OW), 0.01, jnp.float32)
    return _paint_call(vi, const01)
